# baseline (speedup 1.0000x reference)
import functools

import jax
import jax.numpy as jnp
from jax import lax
from jax.experimental import pallas as pl
from jax.experimental.pallas import tpu as pltpu
from jax.experimental.pallas import tpu_sc as plsc

N = 10000
NPAD = 10240
ROWS_PER_SUB = 640
E = 320000
K = 128
NCHUNK = E // K
NCHUNK_PAD = 2560
EL = 100000
ELPAD = 102400
DCH = 2 * (ELPAD // K)
DCH_PAD = 1792
IN_C = 128
HID_C = 128
OUT_C = 64

_MESH = plsc.VectorSubcoreMesh(core_axis_name="c", subcore_axis_name="s")

CHUNKS_PER_CORE = NCHUNK // 2
CHUNKS_PER_SUB = 80


def _edge_start_count(c, s):
    start = c * (16 * CHUNKS_PER_SUB) + s * CHUNKS_PER_SUB
    count = jnp.minimum(CHUNKS_PER_SUB,
                        jnp.maximum(CHUNKS_PER_CORE - s * CHUNKS_PER_SUB, 0))
    return start, count


@functools.partial(
    pl.kernel,
    out_type=jax.ShapeDtypeStruct((2, NPAD, 16), jnp.float32),
    mesh=_MESH,
    compiler_params=pltpu.CompilerParams(use_tc_tiling_on_sc=False),
    scratch_types=[
        pltpu.VMEM((CHUNKS_PER_SUB, K), jnp.int32),
        pltpu.VMEM((K, 16), jnp.float32),
        pltpu.VMEM((ROWS_PER_SUB, 16), jnp.float32),
        pltpu.VMEM_SHARED((NPAD, 16), jnp.float32),
    ],
)
def _sc_degree(edges_hbm, out_hbm, idx_v, ones_v, zero_v, acc_sp):
    c = lax.axis_index("c")
    s = lax.axis_index("s")
    lane = lax.iota(jnp.int32, 16)
    onehot = jnp.where(lane == 0, 1.0, 0.0).astype(jnp.float32)
    zvec = jnp.zeros((16,), jnp.float32)

    @pl.loop(0, K)
    def _(r):
        ones_v[r, :] = onehot

    @pl.loop(0, ROWS_PER_SUB)
    def _(r):
        zero_v[r, :] = zvec

    pltpu.sync_copy(zero_v, acc_sp.at[pl.ds(s * ROWS_PER_SUB, ROWS_PER_SUB)])

    start, count = _edge_start_count(c, s)
    pltpu.sync_copy(edges_hbm.at[1].at[pl.ds(start, CHUNKS_PER_SUB)], idx_v)
    plsc.subcore_barrier()

    @pl.loop(0, CHUNKS_PER_SUB)
    def _(i):
        @pl.when(i < count)
        def _():
            pltpu.sync_copy(ones_v, acc_sp.at[idx_v.at[i]], add=True)

    plsc.subcore_barrier()
    pltpu.sync_copy(
        acc_sp.at[pl.ds(s * ROWS_PER_SUB, ROWS_PER_SUB)],
        out_hbm.at[c].at[pl.ds(s * ROWS_PER_SUB, ROWS_PER_SUB)],
    )


def _make_sc_scatter(C):
    cp = (pltpu.CompilerParams(use_tc_tiling_on_sc=False)
          if C % 128 else None)
    halves = 2 if C == 128 else 1
    idx_rows = CHUNKS_PER_SUB // halves

    @functools.partial(
        pl.kernel,
        out_type=jax.ShapeDtypeStruct((2, NPAD, C), jnp.float32),
        mesh=_MESH,
        compiler_params=cp,
        scratch_types=[
            pltpu.VMEM((idx_rows, K), jnp.int32),
            pltpu.VMEM((idx_rows, K), jnp.int32),
            pltpu.VMEM((K, C), jnp.float32),
            pltpu.VMEM((K, C), jnp.float32),
            pltpu.VMEM_SHARED((NPAD, C), jnp.float32),
            pltpu.SemaphoreType.DMA,
            pltpu.SemaphoreType.DMA,
        ],
    )
    def _sc_scatter(g_hbm, edges_hbm, out_hbm, src_v, dst_v, rows0,
                    rows1, acc_sp, sem0, sem1):
        c = lax.axis_index("c")
        s = lax.axis_index("s")

        @pl.when(s < 15)
        def _():
            pltpu.sync_copy(
                g_hbm.at[pl.ds(s * ROWS_PER_SUB, ROWS_PER_SUB)],
                acc_sp.at[pl.ds(s * ROWS_PER_SUB, ROWS_PER_SUB)],
            )

        @pl.when(s == 15)
        def _():
            pltpu.sync_copy(
                g_hbm.at[pl.ds(15 * ROWS_PER_SUB, N - 15 * ROWS_PER_SUB)],
                acc_sp.at[pl.ds(15 * ROWS_PER_SUB, N - 15 * ROWS_PER_SUB)],
            )

        start, count = _edge_start_count(c, s)

        for h in range(halves):
            base = h * idx_rows
            pltpu.sync_copy(
                edges_hbm.at[0].at[pl.ds(start + base, idx_rows)], src_v)
            pltpu.sync_copy(
                edges_hbm.at[1].at[pl.ds(start + base, idx_rows)], dst_v)
            pltpu.async_copy(g_hbm.at[src_v.at[0]], rows0, sem0)
            if h == 0:
                plsc.subcore_barrier()
            hcount = jnp.minimum(idx_rows,
                                 jnp.maximum(count - base, 0))

            @pl.loop(0, idx_rows, step=2)
            def _(i):
                @pl.when(i < hcount)
                def _():
                    pltpu.async_copy(g_hbm.at[src_v.at[i + 1]], rows1, sem1)
                    pltpu.make_async_copy(
                        g_hbm.at[src_v.at[i]], rows0, sem0).wait()
                    pltpu.sync_copy(rows0, acc_sp.at[dst_v.at[i]], add=True)

                    @pl.when(i + 2 < hcount)
                    def _():
                        pltpu.async_copy(
                            g_hbm.at[src_v.at[i + 2]], rows0, sem0)

                    pltpu.make_async_copy(
                        g_hbm.at[src_v.at[i + 1]], rows1, sem1).wait()
                    pltpu.sync_copy(rows1, acc_sp.at[dst_v.at[i + 1]],
                                    add=True)

        plsc.subcore_barrier()
        pltpu.sync_copy(
            acc_sp.at[pl.ds(s * ROWS_PER_SUB, ROWS_PER_SUB)],
            out_hbm.at[c].at[pl.ds(s * ROWS_PER_SUB, ROWS_PER_SUB)],
        )

    return _sc_scatter


_sc_scatter_hid = _make_sc_scatter(HID_C)
_sc_scatter_out = _make_sc_scatter(OUT_C)


@functools.partial(
    pl.kernel,
    out_type=jax.ShapeDtypeStruct((DCH, K, OUT_C), jnp.float32),
    mesh=_MESH,
    compiler_params=pltpu.CompilerParams(use_tc_tiling_on_sc=False),
    scratch_types=[
        pltpu.VMEM((56, K), jnp.int32),
        pltpu.VMEM((K, OUT_C), jnp.float32),
        pltpu.VMEM((K, OUT_C), jnp.float32),
        pltpu.SemaphoreType.DMA,
        pltpu.SemaphoreType.DMA,
    ],
)
def _sc_decode_gather(z_hbm, idx_hbm, out_hbm, idx_v, rows0, rows1, sem0, sem1):
    c = lax.axis_index("c")
    s = lax.axis_index("s")
    w = c * 16 + s
    start = w * 56
    count = jnp.minimum(56, jnp.maximum(DCH - start, 0))
    pltpu.sync_copy(idx_hbm.at[pl.ds(start, 56)], idx_v)

    @pl.when(count > 0)
    def _():
        pltpu.async_copy(z_hbm.at[idx_v.at[0]], rows0, sem0)

    @pl.loop(0, 56, step=2)
    def _(i):
        @pl.when(i < count)
        def _():
            pltpu.async_copy(z_hbm.at[idx_v.at[i + 1]], rows1, sem1)
            pltpu.make_async_copy(
                z_hbm.at[idx_v.at[i]], rows0, sem0).wait()
            pltpu.sync_copy(rows0, out_hbm.at[start + i])

            @pl.when(i + 2 < count)
            def _():
                pltpu.async_copy(z_hbm.at[idx_v.at[i + 2]], rows0, sem0)

            pltpu.make_async_copy(
                z_hbm.at[idx_v.at[i + 1]], rows1, sem1).wait()
            pltpu.sync_copy(rows1, out_hbm.at[start + i + 1])


BN = 1000


def _mm_body(x_ref, w_ref, o_ref):
    o_ref[...] = jnp.dot(
        x_ref[...], w_ref[...],
        preferred_element_type=jnp.float32,
        precision=lax.Precision.HIGHEST,
    )


def _tc_matmul(x, w):
    m, kdim = x.shape
    n = w.shape[1]
    return pl.pallas_call(
        _mm_body,
        grid=(m // BN,),
        in_specs=[
            pl.BlockSpec((BN, kdim), lambda i: (i, 0)),
            pl.BlockSpec((kdim, n), lambda i: (0, 0)),
        ],
        out_specs=pl.BlockSpec((BN, n), lambda i: (i, 0)),
        out_shape=jax.ShapeDtypeStruct((m, n), jnp.float32),
    )(x, w)


def _dinv_of(deg_ref):
    deg = deg_ref[0, :, 0:1] + deg_ref[1, :, 0:1] + 1.0
    return lax.rsqrt(deg)


def _scale_body(deg_ref, h_ref, o_ref):
    o_ref[...] = h_ref[...] * _dinv_of(deg_ref)


def _tc_scale(deg, h):
    return pl.pallas_call(
        _scale_body,
        grid=(N // BN,),
        in_specs=[
            pl.BlockSpec((2, BN, 16), lambda i: (0, i, 0)),
            pl.BlockSpec((BN, HID_C), lambda i: (i, 0)),
        ],
        out_specs=pl.BlockSpec((BN, HID_C), lambda i: (i, 0)),
        out_shape=jax.ShapeDtypeStruct((N, HID_C), jnp.float32),
    )(deg, h)


def _layer2_body(acc_ref, g1_ref, deg_ref, b1_ref, w2_ref, o_ref):
    dinv = _dinv_of(deg_ref)
    ssum = acc_ref[0] + acc_ref[1] - g1_ref[...]
    h = jnp.maximum(ssum * dinv + b1_ref[...], 0.0)
    o_ref[...] = (
        jnp.dot(h, w2_ref[...], preferred_element_type=jnp.float32,
                precision=lax.Precision.HIGHEST)
        * dinv
    )


def _tc_layer2(acc1, g1, deg, b1, w2):
    return pl.pallas_call(
        _layer2_body,
        grid=(N // BN,),
        in_specs=[
            pl.BlockSpec((2, BN, HID_C), lambda i: (0, i, 0)),
            pl.BlockSpec((BN, HID_C), lambda i: (i, 0)),
            pl.BlockSpec((2, BN, 16), lambda i: (0, i, 0)),
            pl.BlockSpec((HID_C,), lambda i: (0,)),
            pl.BlockSpec((HID_C, OUT_C), lambda i: (0, 0)),
        ],
        out_specs=pl.BlockSpec((BN, OUT_C), lambda i: (i, 0)),
        out_shape=jax.ShapeDtypeStruct((N, OUT_C), jnp.float32),
    )(acc1, g1, deg, b1, w2)


def _zfinal_body(acc_ref, g2_ref, deg_ref, b2_ref, o_ref):
    dinv = _dinv_of(deg_ref)
    o_ref[...] = (acc_ref[0] + acc_ref[1] - g2_ref[...]) * dinv + b2_ref[...]


def _tc_zfinal(acc2, g2, deg, b2):
    return pl.pallas_call(
        _zfinal_body,
        grid=(N // BN,),
        in_specs=[
            pl.BlockSpec((2, BN, OUT_C), lambda i: (0, i, 0)),
            pl.BlockSpec((BN, OUT_C), lambda i: (i, 0)),
            pl.BlockSpec((2, BN, 16), lambda i: (0, i, 0)),
            pl.BlockSpec((OUT_C,), lambda i: (0,)),
        ],
        out_specs=pl.BlockSpec((BN, OUT_C), lambda i: (i, 0)),
        out_shape=jax.ShapeDtypeStruct((N, OUT_C), jnp.float32),
    )(acc2, g2, deg, b2)


DEC_BN = 2560


def _decode_body(s_ref, d_ref, o_ref):
    p = s_ref[...] * d_ref[...]
    lane = lax.broadcasted_iota(jnp.int32, p.shape, 1)
    lo = jnp.sum(jnp.where(lane < OUT_C, p, 0.0), axis=1)
    hi = jnp.sum(jnp.where(lane >= OUT_C, p, 0.0), axis=1)
    o_ref[...] = jax.nn.sigmoid(jnp.stack([lo, hi], axis=0))


def _tc_decode(pairs_view):
    ngrid = (ELPAD // 2) // DEC_BN
    return pl.pallas_call(
        _decode_body,
        grid=(ngrid,),
        in_specs=[
            pl.BlockSpec((DEC_BN, 128), lambda i: (i, 0)),
            pl.BlockSpec((DEC_BN, 128), lambda i: (i + ngrid, 0)),
        ],
        out_specs=pl.BlockSpec((2, DEC_BN), lambda i: (0, i)),
        out_shape=jax.ShapeDtypeStruct((2, ELPAD // 2), jnp.float32),
    )(pairs_view, pairs_view)


def kernel(x, edge_index, edge_label_index, W1, b1, W2, b2):
    half_pad = NCHUNK_PAD // 2 - CHUNKS_PER_CORE
    edges = jnp.pad(
        edge_index.reshape(2, 2, CHUNKS_PER_CORE, K),
        ((0, 0), (0, 0), (0, half_pad), (0, 0))).reshape(2, NCHUNK_PAD, K)
    eli = jnp.concatenate(
        [jnp.pad(edge_label_index[0], (0, ELPAD - EL)),
         jnp.pad(edge_label_index[1], (0, ELPAD - EL))]
    ).reshape(DCH, K)
    eli = jnp.pad(eli, ((0, DCH_PAD - DCH), (0, 0)))

    deg = _sc_degree(edges)
    h1 = _tc_matmul(x, W1)
    g1 = _tc_scale(deg, h1)
    acc1 = _sc_scatter_hid(g1, edges)
    g2 = _tc_layer2(acc1, g1, deg, b1, W2)
    acc2 = _sc_scatter_out(g2, edges)
    z = _tc_zfinal(acc2, g2, deg, b2)
    pairs_view = _sc_decode_gather(z, eli).reshape(ELPAD, 128)
    lohi = _tc_decode(pairs_view)
    logits = jnp.stack([lohi[0], lohi[1]], axis=1).reshape(ELPAD)
    return logits[:EL]


# speedup vs baseline: 1.3883x; 1.3883x over previous
import functools

import jax
import jax.numpy as jnp
import numpy as np
from jax import lax
from jax.experimental import pallas as pl
from jax.experimental.pallas import tpu as pltpu
from jax.experimental.pallas import tpu_sc as plsc

N = 10000
NPAD = 10240
ROWS_PER_SUB = 640
E = 320000
K = 128
NCHUNK = E // K
NCHUNK_PAD = 2560
EL = 100000
ELPAD = 102400
DCH = 2 * (ELPAD // K)
DCH_PAD = 1792
IN_C = 128
HID_C = 128
OUT_C = 64

_MESH = plsc.VectorSubcoreMesh(core_axis_name="c", subcore_axis_name="s")

CHUNKS_PER_CORE = NCHUNK // 2
CHUNKS_PER_SUB = 80


def _edge_start_count(c, s):
    start = c * (16 * CHUNKS_PER_SUB) + s * CHUNKS_PER_SUB
    count = jnp.minimum(CHUNKS_PER_SUB,
                        jnp.maximum(CHUNKS_PER_CORE - s * CHUNKS_PER_SUB, 0))
    return start, count


@functools.partial(
    pl.kernel,
    out_type=jax.ShapeDtypeStruct((2, NPAD, 16), jnp.float32),
    mesh=_MESH,
    compiler_params=pltpu.CompilerParams(use_tc_tiling_on_sc=False),
    scratch_types=[
        pltpu.VMEM((CHUNKS_PER_SUB, K), jnp.int32),
        pltpu.VMEM((K, 16), jnp.float32),
        pltpu.VMEM((ROWS_PER_SUB, 16), jnp.float32),
        pltpu.VMEM_SHARED((NPAD, 16), jnp.float32),
    ],
)
def _sc_degree(edges_hbm, out_hbm, idx_v, ones_v, zero_v, acc_sp):
    c = lax.axis_index("c")
    s = lax.axis_index("s")
    lane = lax.iota(jnp.int32, 16)
    onehot = jnp.where(lane == 0, 1.0, 0.0).astype(jnp.float32)
    zvec = jnp.zeros((16,), jnp.float32)

    @pl.loop(0, K)
    def _(r):
        ones_v[r, :] = onehot

    @pl.loop(0, ROWS_PER_SUB)
    def _(r):
        zero_v[r, :] = zvec

    pltpu.sync_copy(zero_v, acc_sp.at[pl.ds(s * ROWS_PER_SUB, ROWS_PER_SUB)])

    start, count = _edge_start_count(c, s)
    pltpu.sync_copy(edges_hbm.at[1].at[pl.ds(start, CHUNKS_PER_SUB)], idx_v)
    plsc.subcore_barrier()

    @pl.loop(0, CHUNKS_PER_SUB)
    def _(i):
        @pl.when(i < count)
        def _():
            pltpu.sync_copy(ones_v, acc_sp.at[idx_v.at[i]], add=True)

    plsc.subcore_barrier()
    pltpu.sync_copy(
        acc_sp.at[pl.ds(s * ROWS_PER_SUB, ROWS_PER_SUB)],
        out_hbm.at[c].at[pl.ds(s * ROWS_PER_SUB, ROWS_PER_SUB)],
    )


def _make_sc_scatter(C):
    cp = (pltpu.CompilerParams(use_tc_tiling_on_sc=False)
          if C % 128 else None)
    halves = 2 if C == 128 else 1
    idx_rows = CHUNKS_PER_SUB // halves

    @functools.partial(
        pl.kernel,
        out_type=jax.ShapeDtypeStruct((2, NPAD, C), jnp.float32),
        mesh=_MESH,
        compiler_params=cp,
        scratch_types=[
            pltpu.VMEM((idx_rows, K), jnp.int32),
            pltpu.VMEM((idx_rows, K), jnp.int32),
            pltpu.VMEM((K, C), jnp.float32),
            pltpu.VMEM((K, C), jnp.float32),
            pltpu.VMEM_SHARED((NPAD, C), jnp.float32),
            pltpu.SemaphoreType.DMA,
            pltpu.SemaphoreType.DMA,
        ],
    )
    def _sc_scatter(g_hbm, edges_hbm, out_hbm, src_v, dst_v, rows0,
                    rows1, acc_sp, sem0, sem1):
        c = lax.axis_index("c")
        s = lax.axis_index("s")

        @pl.when(s < 15)
        def _():
            pltpu.sync_copy(
                g_hbm.at[pl.ds(s * ROWS_PER_SUB, ROWS_PER_SUB)],
                acc_sp.at[pl.ds(s * ROWS_PER_SUB, ROWS_PER_SUB)],
            )

        @pl.when(s == 15)
        def _():
            pltpu.sync_copy(
                g_hbm.at[pl.ds(15 * ROWS_PER_SUB, N - 15 * ROWS_PER_SUB)],
                acc_sp.at[pl.ds(15 * ROWS_PER_SUB, N - 15 * ROWS_PER_SUB)],
            )

        start, count = _edge_start_count(c, s)

        for h in range(halves):
            base = h * idx_rows
            pltpu.sync_copy(
                edges_hbm.at[0].at[pl.ds(start + base, idx_rows)], src_v)
            pltpu.sync_copy(
                edges_hbm.at[1].at[pl.ds(start + base, idx_rows)], dst_v)
            pltpu.async_copy(g_hbm.at[src_v.at[0]], rows0, sem0)
            if h == 0:
                plsc.subcore_barrier()
            hcount = jnp.minimum(idx_rows,
                                 jnp.maximum(count - base, 0))

            @pl.loop(0, idx_rows, step=2)
            def _(i):
                @pl.when(i < hcount)
                def _():
                    pltpu.async_copy(g_hbm.at[src_v.at[i + 1]], rows1, sem1)
                    pltpu.make_async_copy(
                        g_hbm.at[src_v.at[i]], rows0, sem0).wait()
                    pltpu.sync_copy(rows0, acc_sp.at[dst_v.at[i]], add=True)

                    @pl.when(i + 2 < hcount)
                    def _():
                        pltpu.async_copy(
                            g_hbm.at[src_v.at[i + 2]], rows0, sem0)

                    pltpu.make_async_copy(
                        g_hbm.at[src_v.at[i + 1]], rows1, sem1).wait()
                    pltpu.sync_copy(rows1, acc_sp.at[dst_v.at[i + 1]],
                                    add=True)

        plsc.subcore_barrier()
        pltpu.sync_copy(
            acc_sp.at[pl.ds(s * ROWS_PER_SUB, ROWS_PER_SUB)],
            out_hbm.at[c].at[pl.ds(s * ROWS_PER_SUB, ROWS_PER_SUB)],
        )

    return _sc_scatter


_sc_scatter_hid = _make_sc_scatter(HID_C)
_sc_scatter_out = _make_sc_scatter(OUT_C)


@functools.partial(
    pl.kernel,
    out_type=jax.ShapeDtypeStruct((DCH, K, OUT_C), jnp.float32),
    mesh=_MESH,
    compiler_params=pltpu.CompilerParams(use_tc_tiling_on_sc=False),
    scratch_types=[
        pltpu.VMEM((56, K), jnp.int32),
        pltpu.VMEM((K, OUT_C), jnp.float32),
        pltpu.VMEM((K, OUT_C), jnp.float32),
        pltpu.SemaphoreType.DMA,
        pltpu.SemaphoreType.DMA,
    ],
)
def _sc_decode_gather(z_hbm, idx_hbm, out_hbm, idx_v, rows0, rows1, sem0, sem1):
    c = lax.axis_index("c")
    s = lax.axis_index("s")
    w = c * 16 + s
    start = w * 56
    count = jnp.minimum(56, jnp.maximum(DCH - start, 0))
    pltpu.sync_copy(idx_hbm.at[pl.ds(start, 56)], idx_v)

    @pl.when(count > 0)
    def _():
        pltpu.async_copy(z_hbm.at[idx_v.at[0]], rows0, sem0)

    @pl.loop(0, 56, step=2)
    def _(i):
        @pl.when(i < count)
        def _():
            pltpu.async_copy(z_hbm.at[idx_v.at[i + 1]], rows1, sem1)
            pltpu.make_async_copy(
                z_hbm.at[idx_v.at[i]], rows0, sem0).wait()
            pltpu.sync_copy(rows0, out_hbm.at[start + i])

            @pl.when(i + 2 < count)
            def _():
                pltpu.async_copy(z_hbm.at[idx_v.at[i + 2]], rows0, sem0)

            pltpu.make_async_copy(
                z_hbm.at[idx_v.at[i + 1]], rows1, sem1).wait()
            pltpu.sync_copy(rows1, out_hbm.at[start + i + 1])


BN = 1000


def _mm_body(x_ref, w_ref, o_ref):
    o_ref[...] = jnp.dot(
        x_ref[...], w_ref[...],
        preferred_element_type=jnp.float32,
        precision=lax.Precision.HIGHEST,
    )


def _tc_matmul(x, w):
    m, kdim = x.shape
    n = w.shape[1]
    return pl.pallas_call(
        _mm_body,
        grid=(m // BN,),
        in_specs=[
            pl.BlockSpec((BN, kdim), lambda i: (i, 0)),
            pl.BlockSpec((kdim, n), lambda i: (0, 0)),
        ],
        out_specs=pl.BlockSpec((BN, n), lambda i: (i, 0)),
        out_shape=jax.ShapeDtypeStruct((m, n), jnp.float32),
    )(x, w)


def _dinv_of(deg_ref):
    deg = deg_ref[0, :, 0:1] + deg_ref[1, :, 0:1] + 1.0
    return lax.rsqrt(deg)


def _scale_body(deg_ref, h_ref, o_ref):
    o_ref[...] = h_ref[...] * _dinv_of(deg_ref)


def _tc_scale(deg, h):
    return pl.pallas_call(
        _scale_body,
        grid=(N // BN,),
        in_specs=[
            pl.BlockSpec((2, BN, 16), lambda i: (0, i, 0)),
            pl.BlockSpec((BN, HID_C), lambda i: (i, 0)),
        ],
        out_specs=pl.BlockSpec((BN, HID_C), lambda i: (i, 0)),
        out_shape=jax.ShapeDtypeStruct((N, HID_C), jnp.float32),
    )(deg, h)


def _layer2_body(acc_ref, g1_ref, deg_ref, b1_ref, w2_ref, o_ref):
    dinv = _dinv_of(deg_ref)
    ssum = acc_ref[0] + acc_ref[1] - g1_ref[...]
    h = jnp.maximum(ssum * dinv + b1_ref[...], 0.0)
    o_ref[...] = (
        jnp.dot(h, w2_ref[...], preferred_element_type=jnp.float32,
                precision=lax.Precision.HIGHEST)
        * dinv
    )


def _tc_layer2(acc1, g1, deg, b1, w2):
    return pl.pallas_call(
        _layer2_body,
        grid=(N // BN,),
        in_specs=[
            pl.BlockSpec((2, BN, HID_C), lambda i: (0, i, 0)),
            pl.BlockSpec((BN, HID_C), lambda i: (i, 0)),
            pl.BlockSpec((2, BN, 16), lambda i: (0, i, 0)),
            pl.BlockSpec((HID_C,), lambda i: (0,)),
            pl.BlockSpec((HID_C, OUT_C), lambda i: (0, 0)),
        ],
        out_specs=pl.BlockSpec((BN, OUT_C), lambda i: (i, 0)),
        out_shape=jax.ShapeDtypeStruct((N, OUT_C), jnp.float32),
    )(acc1, g1, deg, b1, w2)


def _zfinal_body(acc_ref, g2_ref, deg_ref, b2_ref, o_ref):
    dinv = _dinv_of(deg_ref)
    o_ref[...] = (acc_ref[0] + acc_ref[1] - g2_ref[...]) * dinv + b2_ref[...]


def _tc_zfinal(acc2, g2, deg, b2):
    return pl.pallas_call(
        _zfinal_body,
        grid=(N // BN,),
        in_specs=[
            pl.BlockSpec((2, BN, OUT_C), lambda i: (0, i, 0)),
            pl.BlockSpec((BN, OUT_C), lambda i: (i, 0)),
            pl.BlockSpec((2, BN, 16), lambda i: (0, i, 0)),
            pl.BlockSpec((OUT_C,), lambda i: (0,)),
        ],
        out_specs=pl.BlockSpec((BN, OUT_C), lambda i: (i, 0)),
        out_shape=jax.ShapeDtypeStruct((N, OUT_C), jnp.float32),
    )(acc2, g2, deg, b2)


DEC_BN = 2560


def _decode_body(s_ref, d_ref, o_ref):
    p = s_ref[...] * d_ref[...]
    lane = lax.broadcasted_iota(jnp.int32, p.shape, 1)
    lo = jnp.sum(jnp.where(lane < OUT_C, p, 0.0), axis=1)
    hi = jnp.sum(jnp.where(lane >= OUT_C, p, 0.0), axis=1)
    o_ref[...] = jax.nn.sigmoid(jnp.stack([lo, hi], axis=0))


def _tc_decode(pairs_view):
    ngrid = (ELPAD // 2) // DEC_BN
    return pl.pallas_call(
        _decode_body,
        grid=(ngrid,),
        in_specs=[
            pl.BlockSpec((DEC_BN, 128), lambda i: (i, 0)),
            pl.BlockSpec((DEC_BN, 128), lambda i: (i + ngrid, 0)),
        ],
        out_specs=pl.BlockSpec((2, DEC_BN), lambda i: (0, i)),
        out_shape=jax.ShapeDtypeStruct((2, ELPAD // 2), jnp.float32),
    )(pairs_view, pairs_view)


_PERM_NP = np.zeros((128, 128), np.float32)
for _v in range(128):
    _PERM_NP[_v // 2 + 64 * (_v % 2), _v] = 1.0


def _assemble_body(lo_ref, hi_ref, p_ref, o_ref):
    t = jnp.concatenate([lo_ref[...], hi_ref[...]], axis=1)
    o_ref[...] = jnp.dot(t, p_ref[...], preferred_element_type=jnp.float32,
                         precision=lax.Precision.HIGHEST)


def _tc_assemble(lo64, hi64):
    rows = ELPAD // 128
    perm = jnp.asarray(_PERM_NP)
    return pl.pallas_call(
        _assemble_body,
        grid=(1,),
        in_specs=[
            pl.BlockSpec((rows, 64), lambda i: (0, 0)),
            pl.BlockSpec((rows, 64), lambda i: (0, 0)),
            pl.BlockSpec((128, 128), lambda i: (0, 0)),
        ],
        out_specs=pl.BlockSpec((rows, 128), lambda i: (0, 0)),
        out_shape=jax.ShapeDtypeStruct((rows, 128), jnp.float32),
    )(lo64, hi64, perm)


def kernel(x, edge_index, edge_label_index, W1, b1, W2, b2):
    half_pad = NCHUNK_PAD // 2 - CHUNKS_PER_CORE
    edges = jnp.pad(
        edge_index.reshape(2, 2, CHUNKS_PER_CORE, K),
        ((0, 0), (0, 0), (0, half_pad), (0, 0))).reshape(2, NCHUNK_PAD, K)
    spread = (jnp.arange(ELPAD - EL, dtype=jnp.int32) * 37) % N
    eli = jnp.concatenate(
        [jnp.concatenate([edge_label_index[0], spread]),
         jnp.concatenate([edge_label_index[1], spread])]
    ).reshape(DCH, K)
    eli = jnp.pad(eli, ((0, DCH_PAD - DCH), (0, 0)))

    deg = _sc_degree(edges)
    h1 = _tc_matmul(x, W1)
    g1 = _tc_scale(deg, h1)
    acc1 = _sc_scatter_hid(g1, edges)
    g2 = _tc_layer2(acc1, g1, deg, b1, W2)
    acc2 = _sc_scatter_out(g2, edges)
    z = _tc_zfinal(acc2, g2, deg, b2)
    pairs_view = _sc_decode_gather(z, eli).reshape(ELPAD, 128)
    lohi = _tc_decode(pairs_view)
    lo64 = lohi[0].reshape(ELPAD // 128, 64)
    hi64 = lohi[1].reshape(ELPAD // 128, 64)
    return _tc_assemble(lo64, hi64).reshape(ELPAD)[:EL]
